# baseline (speedup 1.0000x reference)
import jax
import jax.numpy as jnp
from jax import lax
from jax.experimental import pallas as pl
from jax.experimental.pallas import tpu as pltpu
from jax.experimental.pallas import tpu_sc as plsc

NUM_TX = 4
NUM_STREAMS = 2
NUM_SYM = 14
FFT = 2048
PILOT_SYMS = (2, 11)
PILOT_STRIDE = 4
PILOTS_PER_SYM = FFT // PILOT_STRIDE
PILOTS_PER_STREAM = PILOTS_PER_SYM * 2
DATA_PER_STREAM = NUM_SYM * FFT - PILOTS_PER_STREAM
GRID_PER_STREAM = NUM_SYM * FFT

LANES = 16
BLK = 128
PAIR_IN = NUM_STREAMS * DATA_PER_STREAM
PAIR_OUT = NUM_STREAMS * GRID_PER_STREAM
SYM_OUT = NUM_STREAMS * FFT
SYM_DATA = NUM_STREAMS * (FFT - PILOTS_PER_SYM)
N_PILOTS = NUM_TX * NUM_STREAMS * PILOTS_PER_STREAM
N_SLICE_BUFS = 4
PIL_BASE = N_SLICE_BUFS * SYM_DATA
GROUPS = FFT // LANES

_PSYM_DATA_OFF = []
_src = 0
for _s in range(NUM_SYM):
    if _s in PILOT_SYMS:
        _PSYM_DATA_OFF.append(NUM_STREAMS * _src)
        _src += FFT - PILOTS_PER_SYM
    else:
        _src += FFT


def _sc_pilot_rows(flat_in, flat_pilots, *, pairs, pairs_per_worker):
    mesh = plsc.VectorSubcoreMesh(core_axis_name="c", subcore_axis_name="s")
    info = plsc.get_sparse_core_info()
    nc = info.num_cores
    n_psym = len(PILOT_SYMS)

    @pl.kernel(
        mesh=mesh,
        out_type=jax.ShapeDtypeStruct((pairs * n_psym * SYM_OUT,), jnp.float32),
        scratch_types=[
            pltpu.VMEM((PIL_BASE + N_PILOTS,), jnp.float32),
            pltpu.VMEM((2 * SYM_OUT,), jnp.float32),
            pltpu.VMEM((GROUPS * LANES,), jnp.int32),
            pltpu.SemaphoreType.DMA,
            pltpu.SemaphoreType.DMA,
            pltpu.SemaphoreType.DMA,
        ],
        compiler_params=pltpu.CompilerParams(needs_layout_passes=False),
    )
    def pilot_builder(in_hbm, pil_hbm, out_hbm, big_v, row_v, idx_v,
                      sem_in, sem_r0, sem_r1):
        row_sems = (sem_r0, sem_r1)
        wid = lax.axis_index("s") * nc + lax.axis_index("c")
        lanes = lax.iota(jnp.int32, LANES)
        is_pilot_lane = (lanes & (PILOT_STRIDE - 1)) == 0

        pltpu.sync_copy(pil_hbm, big_v.at[pl.ds(PIL_BASE, N_PILOTS)])

        @plsc.parallel_loop(0, GROUPS, unroll=4)
        def build_idx(gt):
            k = gt * LANES + lanes
            h = k - (k >> 2) - 1
            kq = k >> 2
            dpat = ((h >> 7) << 8) + (h & (BLK - 1))
            ppat = ((kq >> 7) << 8) + (kq & (BLK - 1))
            idx_v[pl.ds(gt * LANES, LANES)] = jnp.where(
                is_pilot_lane, ppat, dpat)

        tasks = []
        for u in range(pairs_per_worker):
            for j in range(n_psym):
                tasks.append((u, j))

        def start_slice(t):
            u, j = tasks[t]
            q = wid * pairs_per_worker + u
            return pltpu.async_copy(
                in_hbm.at[pl.ds(q * PAIR_IN + _PSYM_DATA_OFF[j], SYM_DATA)],
                big_v.at[pl.ds((t % N_SLICE_BUFS) * SYM_DATA, SYM_DATA)],
                sem_in)

        slice_descs = {0: start_slice(0), 1: start_slice(1)}
        row_descs = {}
        for t, (u, j) in enumerate(tasks):
            if t + 2 < len(tasks):
                slice_descs[t + 2] = start_slice(t + 2)
            slice_descs.pop(t).wait()

            q = wid * pairs_per_worker + u
            tx = q & (NUM_TX - 1)
            sbase = (t % N_SLICE_BUFS) * SYM_DATA
            rs = t % 2
            if t - 2 in row_descs:
                row_descs.pop(t - 2).wait()
            pil_off = (PIL_BASE + tx * NUM_STREAMS * PILOTS_PER_STREAM
                       + j * NUM_STREAMS * PILOTS_PER_SYM)
            for s in range(NUM_STREAMS):
                offv = jnp.where(
                    is_pilot_lane,
                    jnp.full((LANES,), pil_off + s * BLK, jnp.int32),
                    jnp.full((LANES,), sbase + s * BLK, jnp.int32))
                rbase = rs * SYM_OUT + s * BLK

                @plsc.parallel_loop(0, GROUPS, unroll=8)
                def gather_group(g, offv=offv, rbase=rbase):
                    iv = idx_v[pl.ds(g * LANES, LANES)] + offv
                    dest = rbase + ((g >> 3) << 8) + ((g & 7) << 4)
                    row_v[pl.ds(dest, LANES)] = (
                        plsc.load_gather(big_v, [iv]))

            row_descs[t] = pltpu.async_copy(
                row_v.at[pl.ds(rs * SYM_OUT, SYM_OUT)],
                out_hbm.at[pl.ds((q * n_psym + j) * SYM_OUT, SYM_OUT)],
                row_sems[rs])

        for d in row_descs.values():
            d.wait()

    return pilot_builder(flat_in, flat_pilots)


def _tc_assemble(in3, pil4, *, pairs):
    in_blocks = DATA_PER_STREAM * NUM_STREAMS // BLK
    out_blocks = GRID_PER_STREAM * NUM_STREAMS // BLK
    sym_blocks = SYM_OUT // BLK

    dense_runs = []
    pilot_dst = []
    src = 0
    for s in range(NUM_SYM):
        if s in PILOT_SYMS:
            pilot_dst.append(s * sym_blocks)
            src += (FFT - PILOTS_PER_SYM) * NUM_STREAMS // BLK
        else:
            so, do = src, s * sym_blocks
            if dense_runs and dense_runs[-1][0] + dense_runs[-1][2] == so:
                a, b, n = dense_runs[-1]
                dense_runs[-1] = (a, b, n + sym_blocks)
            else:
                dense_runs.append((so, do, sym_blocks))
            src += sym_blocks

    def body(in_ref, pil_ref, out_ref):
        for so, do, n in dense_runs:
            out_ref[0, pl.ds(do, n), :] = in_ref[0, pl.ds(so, n), :]
        for j, do in enumerate(pilot_dst):
            out_ref[0, pl.ds(do, sym_blocks), :] = pil_ref[0, j]

    return pl.pallas_call(
        body,
        grid=(pairs,),
        in_specs=[
            pl.BlockSpec((1, in_blocks, BLK), lambda i: (i, 0, 0)),
            pl.BlockSpec((1, 2, sym_blocks, BLK), lambda i: (i, 0, 0, 0)),
        ],
        out_specs=pl.BlockSpec((1, out_blocks, BLK), lambda i: (i, 0, 0)),
        out_shape=jax.ShapeDtypeStruct((pairs, out_blocks, BLK), jnp.float32),
    )(in3, pil4)


def kernel(inputs, pilots):
    batch, num_tx, num_streams, dps = inputs.shape
    pairs = batch * num_tx
    flat_in = inputs.reshape(
        batch, num_tx, num_streams, dps // BLK, BLK).transpose(
        0, 1, 3, 2, 4).reshape(-1)
    flat_pil = pilots.reshape(
        num_tx, num_streams, PILOTS_PER_STREAM // BLK, BLK).transpose(
        0, 2, 1, 3).reshape(-1)

    pilot_rows = _sc_pilot_rows(
        flat_in, flat_pil, pairs=pairs, pairs_per_worker=pairs // 32)

    out3 = _tc_assemble(
        flat_in.reshape(pairs, PAIR_IN // BLK, BLK),
        pilot_rows.reshape(pairs, len(PILOT_SYMS), SYM_OUT // BLK, BLK),
        pairs=pairs)

    out = out3.reshape(
        batch, num_tx, NUM_SYM, FFT // BLK, num_streams, BLK).transpose(
        0, 1, 4, 2, 3, 5).reshape(
        batch, num_tx, num_streams, NUM_SYM, FFT)
    return out


# speedup vs baseline: 2.5275x; 2.5275x over previous
import jax
import jax.numpy as jnp
from jax import lax
from jax.experimental import pallas as pl
from jax.experimental.pallas import tpu as pltpu
from jax.experimental.pallas import tpu_sc as plsc

NUM_TX = 4
NUM_STREAMS = 2
NUM_SYM = 14
FFT = 2048
PILOT_SYMS = (2, 11)
PILOT_STRIDE = 4
PILOTS_PER_SYM = FFT // PILOT_STRIDE
PILOTS_PER_STREAM = PILOTS_PER_SYM * 2
DATA_PER_STREAM = NUM_SYM * FFT - PILOTS_PER_STREAM
GRID_PER_STREAM = NUM_SYM * FFT

LANES = 16
BLK = 128
PAIR_IN = NUM_STREAMS * DATA_PER_STREAM
PAIR_OUT = NUM_STREAMS * GRID_PER_STREAM
SYM_OUT = NUM_STREAMS * FFT
SYM_DATA = NUM_STREAMS * (FFT - PILOTS_PER_SYM)
N_PILOTS = NUM_TX * NUM_STREAMS * PILOTS_PER_STREAM
N_SLICE_BUFS = 4
PIL_BASE = N_SLICE_BUFS * SYM_DATA
GROUPS = FFT // LANES

_PSYM_DATA_OFF = []
_src = 0
for _s in range(NUM_SYM):
    if _s in PILOT_SYMS:
        _PSYM_DATA_OFF.append(NUM_STREAMS * _src)
        _src += FFT - PILOTS_PER_SYM
    else:
        _src += FFT


def _sc_pilot_rows(flat_in, flat_pilots, *, pairs, pairs_per_worker):
    mesh = plsc.VectorSubcoreMesh(core_axis_name="c", subcore_axis_name="s")
    info = plsc.get_sparse_core_info()
    nc = info.num_cores
    n_psym = len(PILOT_SYMS)

    @pl.kernel(
        mesh=mesh,
        out_type=jax.ShapeDtypeStruct((pairs * n_psym * SYM_OUT,), jnp.float32),
        scratch_types=[
            pltpu.VMEM((PIL_BASE + N_PILOTS,), jnp.float32),
            pltpu.VMEM((2 * SYM_OUT,), jnp.float32),
            pltpu.VMEM((GROUPS * LANES,), jnp.int32),
            pltpu.SemaphoreType.DMA,
            pltpu.SemaphoreType.DMA,
            pltpu.SemaphoreType.DMA,
        ],
        compiler_params=pltpu.CompilerParams(needs_layout_passes=False),
    )
    def pilot_builder(in_hbm, pil_hbm, out_hbm, big_v, row_v, idx_v,
                      sem_in, sem_r0, sem_r1):
        row_sems = (sem_r0, sem_r1)
        wid = lax.axis_index("s") * nc + lax.axis_index("c")
        lanes = lax.iota(jnp.int32, LANES)
        is_pilot_lane = (lanes & (PILOT_STRIDE - 1)) == 0

        pltpu.sync_copy(pil_hbm, big_v.at[pl.ds(PIL_BASE, N_PILOTS)])

        @plsc.parallel_loop(0, GROUPS, unroll=4)
        def build_idx(gt):
            k = gt * LANES + lanes
            h = k - (k >> 2) - 1
            kq = k >> 2
            dpat = ((h >> 7) << 8) + (h & (BLK - 1))
            ppat = ((kq >> 7) << 8) + (kq & (BLK - 1))
            idx_v[pl.ds(gt * LANES, LANES)] = jnp.where(
                is_pilot_lane, ppat, dpat)

        tasks = []
        for u in range(pairs_per_worker):
            for j in range(n_psym):
                tasks.append((u, j))

        def start_slice(t):
            u, j = tasks[t]
            q = wid * pairs_per_worker + u
            return pltpu.async_copy(
                in_hbm.at[pl.ds(q * PAIR_IN + _PSYM_DATA_OFF[j], SYM_DATA)],
                big_v.at[pl.ds((t % N_SLICE_BUFS) * SYM_DATA, SYM_DATA)],
                sem_in)

        slice_descs = {0: start_slice(0), 1: start_slice(1)}
        row_descs = {}
        for t, (u, j) in enumerate(tasks):
            if t + 2 < len(tasks):
                slice_descs[t + 2] = start_slice(t + 2)
            slice_descs.pop(t).wait()

            q = wid * pairs_per_worker + u
            tx = q & (NUM_TX - 1)
            sbase = (t % N_SLICE_BUFS) * SYM_DATA
            rs = t % 2
            if t - 2 in row_descs:
                row_descs.pop(t - 2).wait()
            pil_off = (PIL_BASE + tx * NUM_STREAMS * PILOTS_PER_STREAM
                       + j * NUM_STREAMS * PILOTS_PER_SYM)
            for s in range(NUM_STREAMS):
                offv = jnp.where(
                    is_pilot_lane,
                    jnp.full((LANES,), pil_off + s * BLK, jnp.int32),
                    jnp.full((LANES,), sbase + s * BLK, jnp.int32))
                rbase = rs * SYM_OUT + s * BLK

                @plsc.parallel_loop(0, GROUPS, unroll=8)
                def gather_group(g, offv=offv, rbase=rbase):
                    iv = idx_v[pl.ds(g * LANES, LANES)] + offv
                    dest = rbase + ((g >> 3) << 8) + ((g & 7) << 4)
                    row_v[pl.ds(dest, LANES)] = (
                        plsc.load_gather(big_v, [iv]))

            row_descs[t] = pltpu.async_copy(
                row_v.at[pl.ds(rs * SYM_OUT, SYM_OUT)],
                out_hbm.at[pl.ds((q * n_psym + j) * SYM_OUT, SYM_OUT)],
                row_sems[rs])

        for d in row_descs.values():
            d.wait()

    return pilot_builder(flat_in, flat_pilots)


def _tc_assemble(in3, pil4, *, pairs):
    in_blocks = DATA_PER_STREAM * NUM_STREAMS // BLK
    out_blocks = GRID_PER_STREAM * NUM_STREAMS // BLK
    sym_blocks = SYM_OUT // BLK

    dense_runs = []
    pilot_dst = []
    src = 0
    for s in range(NUM_SYM):
        if s in PILOT_SYMS:
            pilot_dst.append(s * sym_blocks)
            src += (FFT - PILOTS_PER_SYM) * NUM_STREAMS // BLK
        else:
            so, do = src, s * sym_blocks
            if dense_runs and dense_runs[-1][0] + dense_runs[-1][2] == so:
                a, b, n = dense_runs[-1]
                dense_runs[-1] = (a, b, n + sym_blocks)
            else:
                dense_runs.append((so, do, sym_blocks))
            src += sym_blocks

    pb = 16

    def body(in_ref, pil_ref, out_ref):
        for p in range(pb):
            for so, do, n in dense_runs:
                out_ref[p, pl.ds(do, n), :] = in_ref[p, pl.ds(so, n), :]
            for j, do in enumerate(pilot_dst):
                out_ref[p, pl.ds(do, sym_blocks), :] = pil_ref[p, j]

    return pl.pallas_call(
        body,
        grid=(pairs // pb,),
        in_specs=[
            pl.BlockSpec((pb, in_blocks, BLK), lambda i: (i, 0, 0)),
            pl.BlockSpec((pb, 2, sym_blocks, BLK), lambda i: (i, 0, 0, 0)),
        ],
        out_specs=pl.BlockSpec((pb, out_blocks, BLK), lambda i: (i, 0, 0)),
        out_shape=jax.ShapeDtypeStruct((pairs, out_blocks, BLK), jnp.float32),
    )(in3, pil4)


def kernel(inputs, pilots):
    batch, num_tx, num_streams, dps = inputs.shape
    pairs = batch * num_tx
    flat_in = inputs.reshape(
        batch, num_tx, num_streams, dps // BLK, BLK).transpose(
        0, 1, 3, 2, 4).reshape(-1)
    flat_pil = pilots.reshape(
        num_tx, num_streams, PILOTS_PER_STREAM // BLK, BLK).transpose(
        0, 2, 1, 3).reshape(-1)

    pilot_rows = _sc_pilot_rows(
        flat_in, flat_pil, pairs=pairs, pairs_per_worker=pairs // 32)

    out3 = _tc_assemble(
        flat_in.reshape(pairs, PAIR_IN // BLK, BLK),
        pilot_rows.reshape(pairs, len(PILOT_SYMS), SYM_OUT // BLK, BLK),
        pairs=pairs)

    out = out3.reshape(
        batch, num_tx, NUM_SYM, FFT // BLK, num_streams, BLK).transpose(
        0, 1, 4, 2, 3, 5).reshape(
        batch, num_tx, num_streams, NUM_SYM, FFT)
    return out
